# Initial kernel scaffold; baseline (speedup 1.0000x reference)
import jax
import jax.numpy as jnp
from jax.experimental import pallas as pl


def kernel(x, edge_index, batch, W_init, b_init, W_msg, b_msg, W_ih, W_hh, b_ih, b_hh, W_fc, b_fc):
    raise NotImplementedError("write your pallas kernel here")


# speedup vs baseline: 2.9759x; 2.9759x over previous
import functools

import jax
import jax.numpy as jnp
from jax import lax
from jax.experimental import pallas as pl
from jax.experimental.pallas import tpu as pltpu
from jax.experimental.pallas import tpu_sc as plsc

N = 10000
E = 160000
H = 256
G = 64
HH = 128

NBLK = 1000
NGRID = N // NBLK

NTILES = 16
CHUNK = 80
EC = E // NTILES
NCH = EC // CHUNK
ZCH = N // CHUNK


def _init_body(x_ref, wi_ref, bi_ref, wt_ref, wb_ref, bm_ref, h_ref, t_ref):
    h = jnp.dot(x_ref[...], wi_ref[...], preferred_element_type=jnp.float32)
    h = h + bi_ref[...]
    h_ref[...] = h
    a = jnp.dot(h, wt_ref[...], preferred_element_type=jnp.float32) + bm_ref[...]
    b = jnp.dot(h, wb_ref[...], preferred_element_type=jnp.float32)
    t_ref[0] = a[:, :HH]
    t_ref[1] = a[:, HH:]
    t_ref[2] = b[:, :HH]
    t_ref[3] = b[:, HH:]


def _tc_init(x, W_init, b_init, Wt, Wb, bm):
    return pl.pallas_call(
        _init_body,
        grid=(NGRID,),
        in_specs=[
            pl.BlockSpec((NBLK, H), lambda i: (i, 0)),
            pl.BlockSpec((H, H), lambda i: (0, 0)),
            pl.BlockSpec((1, H), lambda i: (0, 0)),
            pl.BlockSpec((H, H), lambda i: (0, 0)),
            pl.BlockSpec((H, H), lambda i: (0, 0)),
            pl.BlockSpec((1, H), lambda i: (0, 0)),
        ],
        out_specs=[
            pl.BlockSpec((NBLK, H), lambda i: (i, 0)),
            pl.BlockSpec((4, NBLK, HH), lambda i: (0, i, 0)),
        ],
        out_shape=[
            jax.ShapeDtypeStruct((N, H), jnp.float32),
            jax.ShapeDtypeStruct((4, N, HH), jnp.float32),
        ],
    )(x, W_init, b_init, Wt, Wb, bm)


def _gru_body(has_next, args):
    if has_next:
        (agg_ref, h_ref, wih_ref, whh_ref, bih_ref, bhh_ref,
         wt_ref, wb_ref, bm_ref, hn_ref, t_ref) = args
    else:
        (agg_ref, h_ref, wih_ref, whh_ref, bih_ref, bhh_ref, hn_ref) = args
    h = h_ref[...]
    gi = jnp.dot(agg_ref[0], wih_ref[:HH], preferred_element_type=jnp.float32)
    gi = gi + jnp.dot(agg_ref[1], wih_ref[HH:], preferred_element_type=jnp.float32)
    gi = gi + bih_ref[...]
    gh = jnp.dot(h, whh_ref[...], preferred_element_type=jnp.float32) + bhh_ref[...]
    r = jax.nn.sigmoid(gi[:, :H] + gh[:, :H])
    z = jax.nn.sigmoid(gi[:, H:2 * H] + gh[:, H:2 * H])
    nn_ = jnp.tanh(gi[:, 2 * H:] + r * gh[:, 2 * H:])
    hn = (1.0 - z) * nn_ + z * h
    hn_ref[...] = hn
    if has_next:
        a = jnp.dot(hn, wt_ref[...], preferred_element_type=jnp.float32) + bm_ref[...]
        b = jnp.dot(hn, wb_ref[...], preferred_element_type=jnp.float32)
        t_ref[0] = a[:, :HH]
        t_ref[1] = a[:, HH:]
        t_ref[2] = b[:, :HH]
        t_ref[3] = b[:, HH:]


def _tc_gru(agg, h, W_ih, W_hh, b_ih, b_hh, nxt=None):
    has_next = nxt is not None
    in_specs = [
        pl.BlockSpec((2, NBLK, HH), lambda i: (0, i, 0)),
        pl.BlockSpec((NBLK, H), lambda i: (i, 0)),
        pl.BlockSpec((H, 3 * H), lambda i: (0, 0)),
        pl.BlockSpec((H, 3 * H), lambda i: (0, 0)),
        pl.BlockSpec((1, 3 * H), lambda i: (0, 0)),
        pl.BlockSpec((1, 3 * H), lambda i: (0, 0)),
    ]
    args = [agg, h, W_ih, W_hh, b_ih, b_hh]
    out_specs = [pl.BlockSpec((NBLK, H), lambda i: (i, 0))]
    out_shape = [jax.ShapeDtypeStruct((N, H), jnp.float32)]
    if has_next:
        in_specs += [
            pl.BlockSpec((H, H), lambda i: (0, 0)),
            pl.BlockSpec((H, H), lambda i: (0, 0)),
            pl.BlockSpec((1, H), lambda i: (0, 0)),
        ]
        args += list(nxt)
        out_specs.append(pl.BlockSpec((4, NBLK, HH), lambda i: (0, i, 0)))
        out_shape.append(jax.ShapeDtypeStruct((4, N, HH), jnp.float32))
    body = lambda *refs: _gru_body(has_next, refs)
    return pl.pallas_call(
        body, grid=(NGRID,), in_specs=in_specs,
        out_specs=out_specs, out_shape=out_shape,
    )(*args)


def _pool_body(h_ref, b_ref, wfc_ref, bfc_ref, out_ref, acc, cnt):
    i = pl.program_id(0)

    @pl.when(i == 0)
    def _():
        acc[...] = jnp.zeros_like(acc)
        cnt[...] = jnp.zeros_like(cnt)

    s = jnp.dot(h_ref[...], wfc_ref[...], preferred_element_type=jnp.float32)
    onehot = (b_ref[0] == lax.broadcasted_iota(jnp.int32, (G, NBLK), 0))
    onehot = onehot.astype(jnp.float32)
    acc[...] += jnp.dot(onehot, s, preferred_element_type=jnp.float32)
    cnt[...] += jnp.sum(onehot, axis=1, keepdims=True)

    @pl.when(i == pl.num_programs(0) - 1)
    def _():
        out_ref[...] = acc[...] / jnp.maximum(cnt[...], 1.0) + bfc_ref[...]


def _tc_pool(h, batch3, W_fc, b_fc):
    return pl.pallas_call(
        _pool_body,
        grid=(NGRID,),
        in_specs=[
            pl.BlockSpec((NBLK, H), lambda i: (i, 0)),
            pl.BlockSpec((1, 1, NBLK), lambda i: (i, 0, 0)),
            pl.BlockSpec((H, 1), lambda i: (0, 0)),
            pl.BlockSpec((1, 1), lambda i: (0, 0)),
        ],
        out_specs=pl.BlockSpec((G, 1), lambda i: (0, 0)),
        out_shape=jax.ShapeDtypeStruct((G, 1), jnp.float32),
        scratch_shapes=[
            pltpu.VMEM((G, 1), jnp.float32),
            pltpu.VMEM((G, 1), jnp.float32),
        ],
    )(h, batch3, W_fc, b_fc)


@functools.lru_cache(maxsize=None)
def _make_sc_layer():
    mesh = plsc.VectorSubcoreMesh(core_axis_name="c", subcore_axis_name="s",
                                  num_cores=2, num_subcores=NTILES)

    @functools.partial(
        pl.kernel,
        mesh=mesh,
        out_type=jax.ShapeDtypeStruct((2 * N, HH), jnp.float32),
        scratch_types=[
            pltpu.VMEM((CHUNK,), jnp.int32),
            pltpu.VMEM((CHUNK,), jnp.int32),
            pltpu.VMEM((CHUNK,), jnp.int32),
            pltpu.VMEM((CHUNK, HH), jnp.float32),
            pltpu.VMEM((CHUNK, HH), jnp.float32),
            pltpu.VMEM((CHUNK, HH), jnp.float32),
            pltpu.VMEM_SHARED((N, HH), jnp.float32),
            pltpu.SemaphoreType.DMA,
            pltpu.SemaphoreType.DMA,
        ],
    )
    def _sc_layer(t_hbm, idx_hbm, out_hbm, idx_a, idx_b, idx_s,
                  a_buf, b_buf, m_buf, agg, sem_a, sem_b):
        c = lax.axis_index("c")
        s = lax.axis_index("s")

        def zrow(r, carry):
            for j in range(HH // 16):
                m_buf[r, pl.ds(j * 16, 16)] = jnp.zeros((16,), jnp.float32)
            return carry

        lax.fori_loop(0, CHUNK, zrow, 0)

        nz = (ZCH - s + NTILES - 1) // NTILES

        def zch(k, carry):
            cb = s + NTILES * k
            pltpu.sync_copy(m_buf, agg.at[pl.ds(cb * CHUNK, CHUNK)])
            return carry

        lax.fori_loop(0, nz, zch, 0)
        plsc.subcore_barrier()

        base_e = s * EC

        def ch(k, carry):
            off = base_e + k * CHUNK
            pltpu.sync_copy(idx_hbm.at[pl.ds(c * E + off, CHUNK)], idx_a)
            pltpu.sync_copy(idx_hbm.at[pl.ds((2 + c) * E + off, CHUNK)], idx_b)
            pltpu.sync_copy(idx_hbm.at[pl.ds(off, CHUNK)], idx_s)
            ca = pltpu.async_copy(t_hbm.at[idx_a], a_buf, sem_a)
            cb = pltpu.async_copy(t_hbm.at[idx_b], b_buf, sem_b)
            ca.wait()
            cb.wait()

            def erow(r, cc):
                for j in range(HH // 16):
                    sl = pl.ds(j * 16, 16)
                    m_buf[r, sl] = jnp.maximum(a_buf[r, sl] + b_buf[r, sl], 0.0)
                return cc

            lax.fori_loop(0, CHUNK, erow, 0)
            pltpu.sync_copy(m_buf, agg.at[idx_s], add=True)
            return carry

        lax.fori_loop(0, NCH, ch, 0)
        plsc.subcore_barrier()

        def och(k, carry):
            cb = s + NTILES * k
            pltpu.sync_copy(agg.at[pl.ds(cb * CHUNK, CHUNK)], m_buf)
            pltpu.sync_copy(m_buf, out_hbm.at[pl.ds(c * N + cb * CHUNK, CHUNK)])
            return carry

        lax.fori_loop(0, nz, och, 0)

    return _sc_layer


def kernel(x, edge_index, batch, W_init, b_init, W_msg, b_msg,
           W_ih, W_hh, b_ih, b_hh, W_fc, b_fc):
    L = W_msg.shape[0]
    row = edge_index[0]
    col = edge_index[1]
    idx_all = jnp.concatenate([row, row + N, col + 2 * N, col + 3 * N])

    h, T = _tc_init(x, W_init, b_init.reshape(1, H),
                    W_msg[0, :H], W_msg[0, H:], b_msg[0].reshape(1, H))
    for l in range(L):
        agg = _make_sc_layer()(T.reshape(4 * N, HH), idx_all)
        agg = agg.reshape(2, N, HH)
        if l + 1 < L:
            nxt = (W_msg[l + 1, :H], W_msg[l + 1, H:],
                   b_msg[l + 1].reshape(1, H))
            h, T = _tc_gru(agg, h, W_ih[l], W_hh[l],
                           b_ih[l].reshape(1, 3 * H), b_hh[l].reshape(1, 3 * H),
                           nxt=nxt)
        else:
            (h,) = _tc_gru(agg, h, W_ih[l], W_hh[l],
                           b_ih[l].reshape(1, 3 * H), b_hh[l].reshape(1, 3 * H))
    out = _tc_pool(h, batch.reshape(NGRID, 1, NBLK), W_fc,
                   b_fc.reshape(1, 1))
    return out.reshape(G)


# speedup vs baseline: 4.8320x; 1.6237x over previous
import functools

import jax
import jax.numpy as jnp
from jax import lax
from jax.experimental import pallas as pl
from jax.experimental.pallas import tpu as pltpu
from jax.experimental.pallas import tpu_sc as plsc

N = 10000
E = 160000
H = 256
G = 64
HH = 128

NBLK = 1000
NGRID = N // NBLK

NTILES = 16
CHUNK = 80


def _init_body(x_ref, wi_ref, bi_ref, wt_ref, wb_ref, bm_ref, h_ref, t_ref):
    h = jnp.dot(x_ref[...], wi_ref[...], preferred_element_type=jnp.float32)
    h = h + bi_ref[...]
    h_ref[...] = h
    a = jnp.dot(h, wt_ref[...], preferred_element_type=jnp.float32) + bm_ref[...]
    b = jnp.dot(h, wb_ref[...], preferred_element_type=jnp.float32)
    t_ref[0] = a[:, :HH]
    t_ref[1] = a[:, HH:]
    t_ref[2] = b[:, :HH]
    t_ref[3] = b[:, HH:]


def _tc_init(x, W_init, b_init, Wt, Wb, bm):
    return pl.pallas_call(
        _init_body,
        grid=(NGRID,),
        in_specs=[
            pl.BlockSpec((NBLK, H), lambda i: (i, 0)),
            pl.BlockSpec((H, H), lambda i: (0, 0)),
            pl.BlockSpec((1, H), lambda i: (0, 0)),
            pl.BlockSpec((H, H), lambda i: (0, 0)),
            pl.BlockSpec((H, H), lambda i: (0, 0)),
            pl.BlockSpec((1, H), lambda i: (0, 0)),
        ],
        out_specs=[
            pl.BlockSpec((NBLK, H), lambda i: (i, 0)),
            pl.BlockSpec((4, NBLK, HH), lambda i: (0, i, 0)),
        ],
        out_shape=[
            jax.ShapeDtypeStruct((N, H), jnp.float32),
            jax.ShapeDtypeStruct((4, N, HH), jnp.float32),
        ],
    )(x, W_init, b_init, Wt, Wb, bm)


def _gru_body(has_next, args):
    if has_next:
        (agg_ref, h_ref, wih_ref, whh_ref, bih_ref, bhh_ref,
         wt_ref, wb_ref, bm_ref, hn_ref, t_ref) = args
    else:
        (agg_ref, h_ref, wih_ref, whh_ref, bih_ref, bhh_ref, hn_ref) = args
    h = h_ref[...]
    gi = jnp.dot(agg_ref[0], wih_ref[:HH], preferred_element_type=jnp.float32)
    gi = gi + jnp.dot(agg_ref[1], wih_ref[HH:], preferred_element_type=jnp.float32)
    gi = gi + bih_ref[...]
    gh = jnp.dot(h, whh_ref[...], preferred_element_type=jnp.float32) + bhh_ref[...]
    r = jax.nn.sigmoid(gi[:, :H] + gh[:, :H])
    z = jax.nn.sigmoid(gi[:, H:2 * H] + gh[:, H:2 * H])
    nn_ = jnp.tanh(gi[:, 2 * H:] + r * gh[:, 2 * H:])
    hn = (1.0 - z) * nn_ + z * h
    hn_ref[...] = hn
    if has_next:
        a = jnp.dot(hn, wt_ref[...], preferred_element_type=jnp.float32) + bm_ref[...]
        b = jnp.dot(hn, wb_ref[...], preferred_element_type=jnp.float32)
        t_ref[0] = a[:, :HH]
        t_ref[1] = a[:, HH:]
        t_ref[2] = b[:, :HH]
        t_ref[3] = b[:, HH:]


def _tc_gru(agg, h, W_ih, W_hh, b_ih, b_hh, nxt=None):
    has_next = nxt is not None
    in_specs = [
        pl.BlockSpec((2, NBLK, HH), lambda i: (0, i, 0)),
        pl.BlockSpec((NBLK, H), lambda i: (i, 0)),
        pl.BlockSpec((H, 3 * H), lambda i: (0, 0)),
        pl.BlockSpec((H, 3 * H), lambda i: (0, 0)),
        pl.BlockSpec((1, 3 * H), lambda i: (0, 0)),
        pl.BlockSpec((1, 3 * H), lambda i: (0, 0)),
    ]
    args = [agg, h, W_ih, W_hh, b_ih, b_hh]
    out_specs = [pl.BlockSpec((NBLK, H), lambda i: (i, 0))]
    out_shape = [jax.ShapeDtypeStruct((N, H), jnp.float32)]
    if has_next:
        in_specs += [
            pl.BlockSpec((H, H), lambda i: (0, 0)),
            pl.BlockSpec((H, H), lambda i: (0, 0)),
            pl.BlockSpec((1, H), lambda i: (0, 0)),
        ]
        args += list(nxt)
        out_specs.append(pl.BlockSpec((4, NBLK, HH), lambda i: (0, i, 0)))
        out_shape.append(jax.ShapeDtypeStruct((4, N, HH), jnp.float32))
    body = lambda *refs: _gru_body(has_next, refs)
    return pl.pallas_call(
        body, grid=(NGRID,), in_specs=in_specs,
        out_specs=out_specs, out_shape=out_shape,
    )(*args)


def _pool_body(h_ref, b_ref, wfc_ref, bfc_ref, out_ref, acc, cnt):
    i = pl.program_id(0)

    @pl.when(i == 0)
    def _():
        acc[...] = jnp.zeros_like(acc)
        cnt[...] = jnp.zeros_like(cnt)

    s = jnp.dot(h_ref[...], wfc_ref[...], preferred_element_type=jnp.float32)
    onehot = (b_ref[0] == lax.broadcasted_iota(jnp.int32, (G, NBLK), 0))
    onehot = onehot.astype(jnp.float32)
    acc[...] += jnp.dot(onehot, s, preferred_element_type=jnp.float32)
    cnt[...] += jnp.sum(onehot, axis=1, keepdims=True)

    @pl.when(i == pl.num_programs(0) - 1)
    def _():
        out_ref[...] = acc[...] / jnp.maximum(cnt[...], 1.0) + bfc_ref[...]


def _tc_pool(h, batch3, W_fc, b_fc):
    return pl.pallas_call(
        _pool_body,
        grid=(NGRID,),
        in_specs=[
            pl.BlockSpec((NBLK, H), lambda i: (i, 0)),
            pl.BlockSpec((1, 1, NBLK), lambda i: (i, 0, 0)),
            pl.BlockSpec((H, 1), lambda i: (0, 0)),
            pl.BlockSpec((1, 1), lambda i: (0, 0)),
        ],
        out_specs=pl.BlockSpec((G, 1), lambda i: (0, 0)),
        out_shape=jax.ShapeDtypeStruct((G, 1), jnp.float32),
        scratch_shapes=[
            pltpu.VMEM((G, 1), jnp.float32),
            pltpu.VMEM((G, 1), jnp.float32),
        ],
    )(h, batch3, W_fc, b_fc)


NCHT = ((E // CHUNK) // NTILES) & ~1
NPAIR = NCHT // 2
NLEFT = E // CHUNK - NCHT * NTILES
ZCH = N // CHUNK


@functools.lru_cache(maxsize=None)
def _make_sc_layer():
    mesh = plsc.VectorSubcoreMesh(core_axis_name="c", subcore_axis_name="s",
                                  num_cores=2, num_subcores=NTILES)

    idx_t = pltpu.VMEM((CHUNK,), jnp.int32)
    row_t = pltpu.VMEM((CHUNK, HH), jnp.float32)

    @functools.partial(
        pl.kernel,
        mesh=mesh,
        out_type=jax.ShapeDtypeStruct((2 * N, HH), jnp.float32),
        scratch_types=[
            idx_t, idx_t, idx_t, idx_t, row_t, row_t,
            idx_t, idx_t, idx_t, idx_t, row_t, row_t,
            pltpu.VMEM_SHARED((N, HH), jnp.float32),
            pltpu.SemaphoreType.DMA, pltpu.SemaphoreType.DMA,
            pltpu.SemaphoreType.DMA, pltpu.SemaphoreType.DMA,
            pltpu.SemaphoreType.DMA, pltpu.SemaphoreType.DMA,
        ],
    )
    def _sc_layer(t_hbm, idx_hbm, out_hbm,
                  is0, ia0, ib0, iss0, a0, b0,
                  is1, ia1, ib1, iss1, a1, b1,
                  agg, sa0, sb0, sc0, sa1, sb1, sc1):
        c = lax.axis_index("c")
        s = lax.axis_index("s")
        cn = c * N

        def issue(off, is_, ia_, ib_, abuf, bbuf, sa_, sb_):
            pltpu.sync_copy(idx_hbm.at[pl.ds(off, CHUNK)], is_)
            pltpu.sync_copy(idx_hbm.at[pl.ds((1 + c) * E + off, CHUNK)], ib_)
            for j in range(CHUNK // 16):
                sl = pl.ds(j * 16, 16)
                ia_[sl] = is_[sl] + cn
            pltpu.async_copy(t_hbm.at[ia_], abuf, sa_)
            pltpu.async_copy(t_hbm.at[ib_], bbuf, sb_)

        def wait_gathers(ia_, ib_, abuf, bbuf, sa_, sb_):
            pltpu.make_async_copy(t_hbm.at[ia_], abuf, sa_).wait()
            pltpu.make_async_copy(t_hbm.at[ib_], bbuf, sb_).wait()

        def compute(abuf, bbuf):
            def erow(r, cc):
                for j in range(HH // 16):
                    sl = pl.ds(j * 16, 16)
                    abuf[r, sl] = jnp.maximum(abuf[r, sl] + bbuf[r, sl], 0.0)
                return cc
            lax.fori_loop(0, CHUNK, erow, 0)

        def snap_idx(is_, iss_):
            for j in range(CHUNK // 16):
                sl = pl.ds(j * 16, 16)
                iss_[sl] = is_[sl]

        def zrow(r, carry):
            for j in range(HH // 16):
                a0[r, pl.ds(j * 16, 16)] = jnp.zeros((16,), jnp.float32)
            return carry

        lax.fori_loop(0, CHUNK, zrow, 0)
        nz = (ZCH - s + NTILES - 1) // NTILES

        def zch(k, carry):
            cb = s + NTILES * k
            pltpu.sync_copy(a0, agg.at[pl.ds(cb * CHUNK, CHUNK)])
            return carry

        lax.fori_loop(0, nz, zch, 0)
        plsc.subcore_barrier()

        eb = s * (NCHT * CHUNK)
        issue(eb, is0, ia0, ib0, a0, b0, sa0, sb0)

        def pair(g, carry):
            @pl.when(g > 0)
            def _():
                pltpu.make_async_copy(a1, agg.at[iss1], sc1).wait()

            issue(eb + (2 * g + 1) * CHUNK, is1, ia1, ib1, a1, b1, sa1, sb1)
            wait_gathers(ia0, ib0, a0, b0, sa0, sb0)
            compute(a0, b0)
            snap_idx(is0, iss0)
            pltpu.async_copy(a0, agg.at[iss0], sc0, add=True)
            wait_gathers(ia1, ib1, a1, b1, sa1, sb1)
            compute(a1, b1)
            snap_idx(is1, iss1)
            pltpu.async_copy(a1, agg.at[iss1], sc1, add=True)

            @pl.when(g < NPAIR - 1)
            def _():
                pltpu.make_async_copy(a0, agg.at[iss0], sc0).wait()
                issue(eb + (2 * g + 2) * CHUNK, is0, ia0, ib0, a0, b0,
                      sa0, sb0)
            return carry

        lax.fori_loop(0, NPAIR, pair, 0)
        pltpu.make_async_copy(a0, agg.at[iss0], sc0).wait()
        pltpu.make_async_copy(a1, agg.at[iss1], sc1).wait()

        @pl.when(s < NLEFT)
        def _():
            off = (NCHT * NTILES + s) * CHUNK
            issue(off, is0, ia0, ib0, a0, b0, sa0, sb0)
            wait_gathers(ia0, ib0, a0, b0, sa0, sb0)
            compute(a0, b0)
            pltpu.sync_copy(a0, agg.at[is0], add=True)

        plsc.subcore_barrier()

        def och(k, carry):
            cb = s + NTILES * k
            pltpu.sync_copy(agg.at[pl.ds(cb * CHUNK, CHUNK)], a0)
            pltpu.sync_copy(a0, out_hbm.at[pl.ds(c * N + cb * CHUNK, CHUNK)])
            return carry

        lax.fori_loop(0, nz, och, 0)

    return _sc_layer


def kernel(x, edge_index, batch, W_init, b_init, W_msg, b_msg,
           W_ih, W_hh, b_ih, b_hh, W_fc, b_fc):
    L = W_msg.shape[0]
    row = edge_index[0]
    col = edge_index[1]
    idx_all = jnp.concatenate([row, col + 2 * N, col + 3 * N])

    h, T = _tc_init(x, W_init, b_init.reshape(1, H),
                    W_msg[0, :H], W_msg[0, H:], b_msg[0].reshape(1, H))
    for l in range(L):
        agg = _make_sc_layer()(T.reshape(4 * N, HH), idx_all)
        agg = agg.reshape(2, N, HH)
        if l + 1 < L:
            nxt = (W_msg[l + 1, :H], W_msg[l + 1, H:],
                   b_msg[l + 1].reshape(1, H))
            h, T = _tc_gru(agg, h, W_ih[l], W_hh[l],
                           b_ih[l].reshape(1, 3 * H), b_hh[l].reshape(1, 3 * H),
                           nxt=nxt)
        else:
            (h,) = _tc_gru(agg, h, W_ih[l], W_hh[l],
                           b_ih[l].reshape(1, 3 * H), b_hh[l].reshape(1, 3 * H))
    out = _tc_pool(h, batch.reshape(NGRID, 1, NBLK), W_fc,
                   b_fc.reshape(1, 1))
    return out.reshape(G)
